# baseline (speedup 1.0000x reference)
import jax
import jax.numpy as jnp
from jax.experimental import pallas as pl
from jax.experimental.pallas import tpu as pltpu


def _copy_body(j_ref, o_ref, sem):
    cp = pltpu.make_async_copy(j_ref, o_ref, sem)
    cp.start()
    cp.wait()


def kernel(vertices, joints, extra_joints_idxs):
    del vertices, extra_joints_idxs
    n, j, c = joints.shape
    t = joints.transpose(2, 1, 0)
    out_t = pl.pallas_call(
        _copy_body,
        in_specs=[pl.BlockSpec(memory_space=pl.ANY)],
        out_specs=pl.BlockSpec(memory_space=pl.ANY),
        out_shape=jax.ShapeDtypeStruct((c, j, n), joints.dtype),
        scratch_shapes=[pltpu.SemaphoreType.DMA],
    )(t)
    return out_t.transpose(2, 1, 0)


# speedup vs baseline: 24.0754x; 24.0754x over previous
import jax
import jax.numpy as jnp
from jax.experimental import pallas as pl
from jax.experimental.pallas import tpu as pltpu


def _copy_body(j_ref, o_ref):
    o_ref[...] = j_ref[...]


def kernel(vertices, joints, extra_joints_idxs):
    del vertices, extra_joints_idxs
    n, j, c = joints.shape
    t = joints.transpose(2, 1, 0)
    out_t = pl.pallas_call(
        _copy_body,
        in_specs=[pl.BlockSpec((c, j, n), lambda: (0, 0, 0))],
        out_specs=pl.BlockSpec((c, j, n), lambda: (0, 0, 0)),
        out_shape=jax.ShapeDtypeStruct((c, j, n), joints.dtype),
    )(t)
    return out_t.transpose(2, 1, 0)


# speedup vs baseline: 28.8708x; 1.1992x over previous
import jax
import jax.numpy as jnp
from jax.experimental import pallas as pl
from jax.experimental.pallas import tpu as pltpu


def _copy_body(j_ref, o_ref, vmem, in_sems, out_sems):
    c = j_ref.shape[0]
    ins = []
    for i in range(c):
        cp = pltpu.make_async_copy(j_ref.at[i], vmem.at[i], in_sems.at[i])
        cp.start()
        ins.append(cp)
    outs = []
    for i in range(c):
        ins[i].wait()
        cp = pltpu.make_async_copy(vmem.at[i], o_ref.at[i], out_sems.at[i])
        cp.start()
        outs.append(cp)
    for cp in outs:
        cp.wait()


def kernel(vertices, joints, extra_joints_idxs):
    del vertices, extra_joints_idxs
    n, j, c = joints.shape
    t = joints.transpose(2, 1, 0)
    out_t = pl.pallas_call(
        _copy_body,
        in_specs=[pl.BlockSpec(memory_space=pl.ANY)],
        out_specs=pl.BlockSpec(memory_space=pl.ANY),
        out_shape=jax.ShapeDtypeStruct((c, j, n), joints.dtype),
        scratch_shapes=[
            pltpu.VMEM((c, j, n), joints.dtype),
            pltpu.SemaphoreType.DMA((c,)),
            pltpu.SemaphoreType.DMA((c,)),
        ],
    )(t)
    return out_t.transpose(2, 1, 0)


# speedup vs baseline: 29.0221x; 1.0052x over previous
import jax
import jax.numpy as jnp
from jax.experimental import pallas as pl
from jax.experimental.pallas import tpu as pltpu


def _copy_body(j_ref, o_ref, vmem, in_sems, out_sems):
    c, j, n = j_ref.shape
    nsplit = 2
    w = n // nsplit
    chunks = [(i, slice(q * w, (q + 1) * w))
              for i in range(c) for q in range(nsplit)]
    ins = []
    for k, (i, s) in enumerate(chunks):
        cp = pltpu.make_async_copy(j_ref.at[i, :, s], vmem.at[i, :, s], in_sems.at[k])
        cp.start()
        ins.append(cp)
    outs = []
    for k, (i, s) in enumerate(chunks):
        ins[k].wait()
        cp = pltpu.make_async_copy(vmem.at[i, :, s], o_ref.at[i, :, s], out_sems.at[k])
        cp.start()
        outs.append(cp)
    for cp in outs:
        cp.wait()


def kernel(vertices, joints, extra_joints_idxs):
    del vertices, extra_joints_idxs
    n, j, c = joints.shape
    t = joints.transpose(2, 1, 0)
    out_t = pl.pallas_call(
        _copy_body,
        in_specs=[pl.BlockSpec(memory_space=pl.ANY)],
        out_specs=pl.BlockSpec(memory_space=pl.ANY),
        out_shape=jax.ShapeDtypeStruct((c, j, n), joints.dtype),
        scratch_shapes=[
            pltpu.VMEM((c, j, n), joints.dtype),
            pltpu.SemaphoreType.DMA((2 * c,)),
            pltpu.SemaphoreType.DMA((2 * c,)),
        ],
    )(t)
    return out_t.transpose(2, 1, 0)
